# baseline (speedup 1.0000x reference)
import jax
import jax.numpy as jnp
from jax.experimental import pallas as pl
from jax.experimental.pallas import tpu as pltpu

_B, _R, _C = 4, 4096, 2048
_RBLK = 2048
_NR = _R // _RBLK


def _argmin_body(x_ref, o_ref, m_ref, i_ref):
    r = pl.program_id(1)
    v = x_ref[0]
    bm = jnp.min(v, axis=0, keepdims=True)
    iota = jax.lax.broadcasted_iota(jnp.int32, v.shape, 0)
    bidx = jnp.min(jnp.where(v <= bm, iota, _RBLK), axis=0, keepdims=True)
    bidx = bidx + r * _RBLK

    @pl.when(r == 0)
    def _init():
        m_ref[...] = bm
        i_ref[...] = bidx

    @pl.when(r == _NR - 1)
    def _emit():
        take = bm < m_ref[...]
        o_ref[0] = jnp.where(take, bidx, i_ref[...])


def kernel(x):
    out = pl.pallas_call(
        _argmin_body,
        grid=(_B, _NR),
        in_specs=[pl.BlockSpec((1, _RBLK, _C), lambda b, r: (b, r, 0))],
        out_specs=pl.BlockSpec((1, 1, _C), lambda b, r: (b, 0, 0)),
        out_shape=jax.ShapeDtypeStruct((_B, 1, _C), jnp.int32),
        scratch_shapes=[
            pltpu.VMEM((1, _C), jnp.float32),
            pltpu.VMEM((1, _C), jnp.int32),
        ],
        compiler_params=pltpu.CompilerParams(
            dimension_semantics=("parallel", "arbitrary"),
        ),
    )(x)
    return out.reshape(_B, _C).astype(jnp.int64)


# speedup vs baseline: 1.0208x; 1.0208x over previous
import jax
import jax.numpy as jnp
from jax.experimental import pallas as pl
from jax.experimental.pallas import tpu as pltpu

_B, _R, _C = 4, 4096, 2048
_RBLK = 2048
_NR = _R // _RBLK
_CH = 1024


def _argmin_body(x_ref, o_ref, m_ref, i_ref):
    r = pl.program_id(1)
    for c in range(_C // _CH):
        cols = slice(c * _CH, (c + 1) * _CH)

        def scan_body(a, carry):
            amin, aidx = carry
            sl = x_ref[0, pl.ds(a * 8, 8), cols]
            took = sl < amin
            return jnp.minimum(amin, sl), jnp.where(took, a, aidx)

        init = (x_ref[0, 0:8, cols], jnp.zeros((8, _CH), jnp.int32))
        amin, aidx = jax.lax.fori_loop(1, _RBLK // 8, scan_body, init,
                                       unroll=4)

        rows = aidx * 8 + jax.lax.broadcasted_iota(jnp.int32, (8, _CH), 0)
        bm = jnp.min(amin, axis=0, keepdims=True)
        bidx = jnp.min(jnp.where(amin <= bm, rows, _R), axis=0,
                       keepdims=True) + r * _RBLK

        @pl.when(r == 0)
        def _init():
            m_ref[0:1, cols] = bm
            i_ref[0:1, cols] = bidx

        @pl.when(r == _NR - 1)
        def _emit():
            take = bm < m_ref[0:1, cols]
            o_ref[0, 0:1, cols] = jnp.where(take, bidx, i_ref[0:1, cols])


def kernel(x):
    out = pl.pallas_call(
        _argmin_body,
        grid=(_B, _NR),
        in_specs=[pl.BlockSpec((1, _RBLK, _C), lambda b, r: (b, r, 0))],
        out_specs=pl.BlockSpec((1, 1, _C), lambda b, r: (b, 0, 0)),
        out_shape=jax.ShapeDtypeStruct((_B, 1, _C), jnp.int32),
        scratch_shapes=[
            pltpu.VMEM((1, _C), jnp.float32),
            pltpu.VMEM((1, _C), jnp.int32),
        ],
        compiler_params=pltpu.CompilerParams(
            dimension_semantics=("parallel", "arbitrary"),
        ),
    )(x)
    return out.reshape(_B, _C).astype(jnp.int64)
